# baseline (speedup 1.0000x reference)
import functools

import jax
import jax.numpy as jnp
from jax.experimental import pallas as pl
from jax.experimental.pallas import tpu as pltpu

_CHUNK = 80
_NBUF = 8


def _gcn_kernel(x_ref, w_ref, b_ref, adj_hbm, o_hbm, s_ref, buf_ref,
                ostage_ref, sem, osem, *, nchunks):
    nfront = (nchunks + 1) // 2

    def block(i):
        if isinstance(i, int):
            return i // 2 if i % 2 == 0 else nfront + i // 2
        return jnp.where(jax.lax.rem(i, 2) == 0, i // 2, nfront + i // 2)
    def copy(i):
        slot = jax.lax.rem(i, _NBUF)
        return pltpu.make_async_copy(
            adj_hbm.at[pl.ds(block(i) * _CHUNK, _CHUNK), :],
            buf_ref.at[slot],
            sem.at[slot],
        )

    for i in range(_NBUF):
        copy(i).start()

    s_ref[...] = jnp.dot(x_ref[...], w_ref[...],
                         preferred_element_type=jnp.float32)

    def ocopy(i):
        oslot = jax.lax.rem(i, 2) if not isinstance(i, int) else i % 2
        return pltpu.make_async_copy(
            ostage_ref.at[oslot],
            o_hbm.at[pl.ds(block(i) * _CHUNK, _CHUNK), :],
            osem.at[oslot],
        )

    def step(i, slot, oslot, last):
        copy_i = pltpu.make_async_copy(
            adj_hbm.at[pl.ds(block(i) * _CHUNK, _CHUNK), :],
            buf_ref.at[slot],
            sem.at[slot],
        )
        copy_i.wait()

        @pl.when(i >= 2)
        def _():
            ocopy(i - 2).wait()

        ostage_ref[oslot] = jnp.dot(
            buf_ref[slot], s_ref[...],
            preferred_element_type=jnp.float32) + b_ref[...]
        ocopy(i).start()
        if not last:
            j = i + _NBUF

            @pl.when(j < nchunks)
            def _():
                pltpu.make_async_copy(
                    adj_hbm.at[pl.ds(block(j) * _CHUNK, _CHUNK), :],
                    buf_ref.at[slot],
                    sem.at[slot],
                ).start()

    nmain = nchunks - nchunks % _NBUF

    def body(m, carry):
        for town in range(_NBUF):
            i = m * _NBUF + town
            step(i, town, town % 2, last=False)
        return carry

    jax.lax.fori_loop(0, nmain // _NBUF, body, 0, unroll=False)
    for i in range(nmain, nchunks):
        step(i, i % _NBUF, i % 2, last=True)
    ocopy(nchunks - 2).wait()
    ocopy(nchunks - 1).wait()


def kernel(x, adj, W, b):
    n, nfeat = x.shape
    nclass = W.shape[1]
    nchunks = n // _CHUNK

    out = pl.pallas_call(
        functools.partial(_gcn_kernel, nchunks=nchunks),
        in_specs=[
            pl.BlockSpec((n, nfeat), lambda: (0, 0)),
            pl.BlockSpec((nfeat, nclass), lambda: (0, 0)),
            pl.BlockSpec((1, nclass), lambda: (0, 0)),
            pl.BlockSpec(memory_space=pl.ANY),
        ],
        out_specs=pl.BlockSpec(memory_space=pl.ANY),
        out_shape=jax.ShapeDtypeStruct((n, nclass), jnp.float32),
        scratch_shapes=[
            pltpu.VMEM((n, nclass), jnp.float32),
            pltpu.VMEM((_NBUF, _CHUNK, n), jnp.float32),
            pltpu.VMEM((2, _CHUNK, nclass), jnp.float32),
            pltpu.SemaphoreType.DMA((_NBUF,)),
            pltpu.SemaphoreType.DMA((2,)),
        ],
    )(x, W, b.reshape(1, nclass), adj)
    return out


# speedup vs baseline: 1.0012x; 1.0012x over previous
import functools

import jax
import jax.numpy as jnp
from jax.experimental import pallas as pl
from jax.experimental.pallas import tpu as pltpu

_CHUNK = 80
_NBUF = 8


def _gcn_kernel(x_ref, w_ref, b_ref, adj_hbm, o_hbm, s_ref, buf_ref,
                ostage_ref, sem, osem, *, nchunks):
    def copy(i):
        slot = jax.lax.rem(i, _NBUF)
        return pltpu.make_async_copy(
            adj_hbm.at[pl.ds(i * _CHUNK, _CHUNK), :],
            buf_ref.at[slot],
            sem.at[slot],
        )

    for i in range(_NBUF):
        copy(i).start()

    s_ref[...] = jnp.dot(x_ref[...], w_ref[...],
                         preferred_element_type=jnp.float32)

    def ocopy(i):
        oslot = jax.lax.rem(i, 2) if not isinstance(i, int) else i % 2
        return pltpu.make_async_copy(
            ostage_ref.at[oslot],
            o_hbm.at[pl.ds(i * _CHUNK, _CHUNK), :],
            osem.at[oslot],
        )

    def step(i, slot, oslot, last):
        copy_i = pltpu.make_async_copy(
            adj_hbm.at[pl.ds(i * _CHUNK, _CHUNK), :],
            buf_ref.at[slot],
            sem.at[slot],
        )
        copy_i.wait()

        @pl.when(i >= 2)
        def _():
            ocopy(i - 2).wait()

        ostage_ref[oslot] = jnp.dot(
            buf_ref[slot], s_ref[...],
            preferred_element_type=jnp.float32) + b_ref[...]
        ocopy(i).start()
        if not last:
            j = i + _NBUF

            @pl.when(j < nchunks)
            def _():
                pltpu.make_async_copy(
                    adj_hbm.at[pl.ds(j * _CHUNK, _CHUNK), :],
                    buf_ref.at[slot],
                    sem.at[slot],
                ).start()

    nmain = nchunks - nchunks % _NBUF

    def body(m, carry):
        for town in range(_NBUF):
            i = m * _NBUF + town
            step(i, town, town % 2, last=False)
        return carry

    jax.lax.fori_loop(0, nmain // _NBUF, body, 0, unroll=False)
    for i in range(nmain, nchunks):
        step(i, i % _NBUF, i % 2, last=True)
    ocopy(nchunks - 2).wait()
    ocopy(nchunks - 1).wait()


def kernel(x, adj, W, b):
    n, nfeat = x.shape
    nclass = W.shape[1]
    nchunks = n // _CHUNK

    out = pl.pallas_call(
        functools.partial(_gcn_kernel, nchunks=nchunks),
        in_specs=[
            pl.BlockSpec((n, nfeat), lambda: (0, 0)),
            pl.BlockSpec((nfeat, nclass), lambda: (0, 0)),
            pl.BlockSpec((1, nclass), lambda: (0, 0)),
            pl.BlockSpec(memory_space=pl.ANY),
        ],
        out_specs=pl.BlockSpec(memory_space=pl.ANY),
        out_shape=jax.ShapeDtypeStruct((n, nclass), jnp.float32),
        scratch_shapes=[
            pltpu.VMEM((n, nclass), jnp.float32),
            pltpu.VMEM((_NBUF, _CHUNK, n), jnp.float32),
            pltpu.VMEM((2, _CHUNK, nclass), jnp.float32),
            pltpu.SemaphoreType.DMA((_NBUF,)),
            pltpu.SemaphoreType.DMA((2,)),
        ],
    )(x, W, b.reshape(1, nclass), adj)
    return out


# speedup vs baseline: 1.0142x; 1.0130x over previous
import functools

import jax
import jax.numpy as jnp
from jax.experimental import pallas as pl
from jax.experimental.pallas import tpu as pltpu

_CHUNK = 80
_NBUF = 6


def _gcn_kernel(x_ref, w_ref, b_ref, adj_hbm, o_hbm, s_ref, buf_ref,
                ostage_ref, sem, osem, *, nchunks):
    def copy(i):
        slot = jax.lax.rem(i, _NBUF)
        return pltpu.make_async_copy(
            adj_hbm.at[pl.ds(i * _CHUNK, _CHUNK), :],
            buf_ref.at[slot],
            sem.at[slot],
        )

    for i in range(_NBUF):
        copy(i).start()

    s_ref[...] = jnp.dot(x_ref[...], w_ref[...],
                         preferred_element_type=jnp.float32)

    def ocopy(i):
        oslot = jax.lax.rem(i, 2) if not isinstance(i, int) else i % 2
        return pltpu.make_async_copy(
            ostage_ref.at[oslot],
            o_hbm.at[pl.ds(i * _CHUNK, _CHUNK), :],
            osem.at[oslot],
        )

    def step(i, slot, oslot, last):
        copy_i = pltpu.make_async_copy(
            adj_hbm.at[pl.ds(i * _CHUNK, _CHUNK), :],
            buf_ref.at[slot],
            sem.at[slot],
        )
        copy_i.wait()

        @pl.when(i >= 2)
        def _():
            ocopy(i - 2).wait()

        ostage_ref[oslot] = jnp.dot(
            buf_ref[slot], s_ref[...],
            preferred_element_type=jnp.float32) + b_ref[...]
        ocopy(i).start()
        if not last:
            j = i + _NBUF

            @pl.when(j < nchunks)
            def _():
                pltpu.make_async_copy(
                    adj_hbm.at[pl.ds(j * _CHUNK, _CHUNK), :],
                    buf_ref.at[slot],
                    sem.at[slot],
                ).start()

    nmain = nchunks - nchunks % _NBUF

    def body(m, carry):
        for town in range(_NBUF):
            i = m * _NBUF + town
            step(i, town, town % 2, last=False)
        return carry

    jax.lax.fori_loop(0, nmain // _NBUF, body, 0, unroll=False)
    for i in range(nmain, nchunks):
        step(i, i % _NBUF, i % 2, last=True)
    ocopy(nchunks - 2).wait()
    ocopy(nchunks - 1).wait()


def kernel(x, adj, W, b):
    n, nfeat = x.shape
    nclass = W.shape[1]
    nchunks = n // _CHUNK

    out = pl.pallas_call(
        functools.partial(_gcn_kernel, nchunks=nchunks),
        in_specs=[
            pl.BlockSpec((n, nfeat), lambda: (0, 0)),
            pl.BlockSpec((nfeat, nclass), lambda: (0, 0)),
            pl.BlockSpec((1, nclass), lambda: (0, 0)),
            pl.BlockSpec(memory_space=pl.ANY),
        ],
        out_specs=pl.BlockSpec(memory_space=pl.ANY),
        out_shape=jax.ShapeDtypeStruct((n, nclass), jnp.float32),
        scratch_shapes=[
            pltpu.VMEM((n, nclass), jnp.float32),
            pltpu.VMEM((_NBUF, _CHUNK, n), jnp.float32),
            pltpu.VMEM((2, _CHUNK, nclass), jnp.float32),
            pltpu.SemaphoreType.DMA((_NBUF,)),
            pltpu.SemaphoreType.DMA((2,)),
        ],
    )(x, W, b.reshape(1, nclass), adj)
    return out


# speedup vs baseline: 1.0331x; 1.0187x over previous
import functools

import jax
import jax.numpy as jnp
from jax.experimental import pallas as pl
from jax.experimental.pallas import tpu as pltpu

_CHUNK = 80
_NBUF = 4


def _gcn_kernel(x_ref, w_ref, b_ref, adj_hbm, o_hbm, s_ref, buf_ref,
                ostage_ref, sem, osem, *, nchunks):
    def copy(i):
        slot = jax.lax.rem(i, _NBUF)
        return pltpu.make_async_copy(
            adj_hbm.at[pl.ds(i * _CHUNK, _CHUNK), :],
            buf_ref.at[slot],
            sem.at[slot],
        )

    for i in range(_NBUF):
        copy(i).start()

    s_ref[...] = jnp.dot(x_ref[...], w_ref[...],
                         preferred_element_type=jnp.float32)

    def ocopy(i):
        oslot = jax.lax.rem(i, 2) if not isinstance(i, int) else i % 2
        return pltpu.make_async_copy(
            ostage_ref.at[oslot],
            o_hbm.at[pl.ds(i * _CHUNK, _CHUNK), :],
            osem.at[oslot],
        )

    def step(i, slot, oslot, last):
        copy_i = pltpu.make_async_copy(
            adj_hbm.at[pl.ds(i * _CHUNK, _CHUNK), :],
            buf_ref.at[slot],
            sem.at[slot],
        )
        copy_i.wait()

        @pl.when(i >= 2)
        def _():
            ocopy(i - 2).wait()

        ostage_ref[oslot] = jnp.dot(
            buf_ref[slot], s_ref[...],
            preferred_element_type=jnp.float32) + b_ref[...]
        ocopy(i).start()
        if not last:
            j = i + _NBUF

            @pl.when(j < nchunks)
            def _():
                pltpu.make_async_copy(
                    adj_hbm.at[pl.ds(j * _CHUNK, _CHUNK), :],
                    buf_ref.at[slot],
                    sem.at[slot],
                ).start()

    nmain = nchunks - nchunks % _NBUF

    def body(m, carry):
        for town in range(_NBUF):
            i = m * _NBUF + town
            step(i, town, town % 2, last=False)
        return carry

    jax.lax.fori_loop(0, nmain // _NBUF, body, 0, unroll=False)
    for i in range(nmain, nchunks):
        step(i, i % _NBUF, i % 2, last=True)
    ocopy(nchunks - 2).wait()
    ocopy(nchunks - 1).wait()


def kernel(x, adj, W, b):
    n, nfeat = x.shape
    nclass = W.shape[1]
    nchunks = n // _CHUNK

    out = pl.pallas_call(
        functools.partial(_gcn_kernel, nchunks=nchunks),
        in_specs=[
            pl.BlockSpec((n, nfeat), lambda: (0, 0)),
            pl.BlockSpec((nfeat, nclass), lambda: (0, 0)),
            pl.BlockSpec((1, nclass), lambda: (0, 0)),
            pl.BlockSpec(memory_space=pl.ANY),
        ],
        out_specs=pl.BlockSpec(memory_space=pl.ANY),
        out_shape=jax.ShapeDtypeStruct((n, nclass), jnp.float32),
        scratch_shapes=[
            pltpu.VMEM((n, nclass), jnp.float32),
            pltpu.VMEM((_NBUF, _CHUNK, n), jnp.float32),
            pltpu.VMEM((2, _CHUNK, nclass), jnp.float32),
            pltpu.SemaphoreType.DMA((_NBUF,)),
            pltpu.SemaphoreType.DMA((2,)),
        ],
    )(x, W, b.reshape(1, nclass), adj)
    return out
